# speedup vs baseline: 5.7488x; 5.7488x over previous
import functools

import jax
import jax.numpy as jnp
from jax import lax
from jax.experimental import pallas as pl
from jax.experimental.pallas import tpu as pltpu
from jax.experimental.pallas import tpu_sc as plsc

VOCAB_SIZE = 100000
HIDDEN = 128
BATCH = 1024
SEQ_LEN = 200
NTOK = BATCH * SEQ_LEN

NUM_CORES = 2
NUM_SUBCORES = 16
NW = NUM_CORES * NUM_SUBCORES
TOK_PER_W = NTOK // NW
CHUNK = 128
STEPS = TOK_PER_W // CHUNK

_MESH = plsc.VectorSubcoreMesh(core_axis_name="c", subcore_axis_name="s")


@functools.partial(
    pl.kernel,
    out_type=jax.ShapeDtypeStruct((NTOK, HIDDEN), jnp.float32),
    mesh=_MESH,
    scratch_types=[
        pltpu.VMEM((STEPS, CHUNK), jnp.int32),
        pltpu.VMEM((CHUNK, HIDDEN), jnp.float32),
        pltpu.SemaphoreType.DMA,
    ],
)
def _embed_sc(idx_hbm, table_hbm, out_hbm, idx_v, rows_v, gsem):
    wid = lax.axis_index("s") * NUM_CORES + lax.axis_index("c")
    base = wid * TOK_PER_W
    pltpu.sync_copy(idx_hbm.at[wid], idx_v)

    @pl.loop(0, STEPS)
    def _step(s):
        pltpu.async_copy(table_hbm.at[idx_v.at[s]], rows_v, gsem).wait()
        pltpu.sync_copy(rows_v, out_hbm.at[pl.ds(base + s * CHUNK, CHUNK)])


def kernel(input_ids, weight):
    idx = input_ids.reshape(NW, STEPS, CHUNK).astype(jnp.int32)
    out = _embed_sc(idx, weight)
    return out.reshape(BATCH, SEQ_LEN, HIDDEN)


# speedup vs baseline: 7.8903x; 1.3725x over previous
import functools

import jax
import jax.numpy as jnp
from jax import lax
from jax.experimental import pallas as pl
from jax.experimental.pallas import tpu as pltpu
from jax.experimental.pallas import tpu_sc as plsc

VOCAB_SIZE = 100000
HIDDEN = 128
BATCH = 1024
SEQ_LEN = 200
NTOK = BATCH * SEQ_LEN

NUM_CORES = 2
NUM_SUBCORES = 16
NW = NUM_CORES * NUM_SUBCORES
TOK_PER_W = NTOK // NW
CHUNK = 128
STEPS = TOK_PER_W // CHUNK

_MESH = plsc.VectorSubcoreMesh(core_axis_name="c", subcore_axis_name="s")


NBUF = 2


@functools.partial(
    pl.kernel,
    out_type=jax.ShapeDtypeStruct((NTOK, HIDDEN), jnp.float32),
    mesh=_MESH,
    scratch_types=[
        pltpu.VMEM((STEPS, CHUNK), jnp.int32),
        pltpu.VMEM((NBUF, CHUNK, HIDDEN), jnp.float32),
        [pltpu.SemaphoreType.DMA] * NBUF,
        [pltpu.SemaphoreType.DMA] * NBUF,
    ],
)
def _embed_sc(idx_hbm, table_hbm, out_hbm, idx_v, rows_v, gsems, osems):
    wid = lax.axis_index("s") * NUM_CORES + lax.axis_index("c")
    base = wid * TOK_PER_W
    pltpu.sync_copy(idx_hbm.at[wid], idx_v)

    def start_gather(s, b):
        pltpu.async_copy(table_hbm.at[idx_v.at[s]], rows_v.at[b], gsems[b])

    def wait_gather(s, b):
        pltpu.make_async_copy(table_hbm.at[idx_v.at[s]], rows_v.at[b], gsems[b]).wait()

    def out_copy(s, b):
        dst = out_hbm.at[pl.ds(base + s * CHUNK, CHUNK)]
        pltpu.async_copy(rows_v.at[b], dst, osems[b])
        return pltpu.make_async_copy(rows_v.at[b], dst, osems[b])

    for b in range(NBUF):
        start_gather(b, b)

    @pl.loop(0, STEPS - NBUF, step=NBUF)
    def _step(s0):
        for b in range(NBUF):
            s = s0 + b
            wait_gather(s, b)
            out = out_copy(s, b)
            out.wait()
            start_gather(s + NBUF, b)

    for b in range(NBUF):
        s = STEPS - NBUF + b
        wait_gather(s, b)
        out_copy(s, b).wait()


def kernel(input_ids, weight):
    idx = input_ids.reshape(NW, STEPS, CHUNK).astype(jnp.int32)
    out = _embed_sc(idx, weight)
    return out.reshape(BATCH, SEQ_LEN, HIDDEN)
